# baseline (device time: 35869 ns/iter reference)
import jax
import jax.numpy as jnp
from jax import lax
from jax.experimental import pallas as pl
from jax.experimental.pallas import tpu as pltpu

N_DEV = 32
K_CHUNK = 512


def kernel(x, w_mat, scale_x, scale_w):
    m_per, k = x.shape
    n = w_mat.shape[1]
    n_per = n // N_DEV
    n_kc = k // K_CHUNK

    def body(x_ref, w_hbm, sx_ref, sw_ref, out_ref,
             w_buf, acc_ref, comm_ref, recv_ref,
             load_sems, send_sems, recv_sems):
        me = lax.axis_index("i")

        barrier_sem = pltpu.get_barrier_semaphore()
        for d in range(1, N_DEV):
            pl.semaphore_signal(
                barrier_sem, inc=1,
                device_id=((me + d) % N_DEV,),
                device_id_type=pl.DeviceIdType.MESH,
            )

        scale = sx_ref[0] * sw_ref[0]

        def start_load(jj):
            cp = pltpu.make_async_copy(
                w_hbm.at[pl.ds(jj * K_CHUNK, K_CHUNK), :],
                w_buf.at[jj % 2],
                load_sems.at[jj % 2],
            )
            cp.start()
            return cp

        load = start_load(0)
        for jj in range(n_kc):
            nxt = start_load(jj + 1) if jj + 1 < n_kc else None
            load.wait()
            partial = jnp.dot(
                x_ref[:, jj * K_CHUNK:(jj + 1) * K_CHUNK],
                w_buf[jj % 2],
                preferred_element_type=jnp.float32,
                precision=lax.Precision.DEFAULT,
            )
            if jj == 0:
                acc_ref[...] = partial
            else:
                acc_ref[...] = acc_ref[...] + partial
            load = nxt

        y = acc_ref[...] * scale
        y_bf = y.astype(jnp.bfloat16)
        for b in range(N_DEV):
            comm_ref[b] = y_bf[:, b * n_per:(b + 1) * n_per]

            @pl.when(b == me)
            def _(b=b):
                out_ref[pl.ds(me * m_per, m_per), :] = (
                    y[:, b * n_per:(b + 1) * n_per]
                )

        pl.semaphore_wait(barrier_sem, N_DEV - 1)

        sends = []
        for d in range(1, N_DEV):
            tgt = (me + d) % N_DEV
            rdma = pltpu.make_async_remote_copy(
                src_ref=comm_ref.at[tgt],
                dst_ref=recv_ref.at[me],
                send_sem=send_sems.at[d],
                recv_sem=recv_sems.at[me],
                device_id=tgt,
                device_id_type=pl.DeviceIdType.LOGICAL,
            )
            rdma.start()
            sends.append(rdma)

        for d in range(1, N_DEV):
            src = (me - d) % N_DEV
            recv = pltpu.make_async_remote_copy(
                src_ref=comm_ref.at[0],
                dst_ref=recv_ref.at[src],
                send_sem=send_sems.at[0],
                recv_sem=recv_sems.at[src],
                device_id=me,
                device_id_type=pl.DeviceIdType.LOGICAL,
            )
            recv.wait_recv()
            out_ref[pl.ds(src * m_per, m_per), :] = (
                recv_ref[src].astype(jnp.float32)
            )

        for rdma in sends:
            rdma.wait_send()

    return pl.pallas_call(
        body,
        out_shape=jax.ShapeDtypeStruct((N_DEV * m_per, n_per), jnp.float32),
        in_specs=[
            pl.BlockSpec(memory_space=pltpu.VMEM),
            pl.BlockSpec(memory_space=pl.ANY),
            pl.BlockSpec(memory_space=pltpu.SMEM),
            pl.BlockSpec(memory_space=pltpu.SMEM),
        ],
        out_specs=pl.BlockSpec(memory_space=pltpu.VMEM),
        scratch_shapes=[
            pltpu.VMEM((2, K_CHUNK, n), jnp.float32),
            pltpu.VMEM((m_per, n), jnp.float32),
            pltpu.VMEM((N_DEV, m_per, n_per), jnp.bfloat16),
            pltpu.VMEM((N_DEV, m_per, n_per), jnp.bfloat16),
            pltpu.SemaphoreType.DMA((2,)),
            pltpu.SemaphoreType.DMA((N_DEV,)),
            pltpu.SemaphoreType.DMA((N_DEV,)),
        ],
        compiler_params=pltpu.CompilerParams(
            collective_id=0,
            vmem_limit_bytes=100 * 1024 * 1024,
        ),
    )(x, w_mat, scale_x, scale_w)
